# baseline (speedup 1.0000x reference)
import functools

import jax
import jax.numpy as jnp
from jax import lax
from jax.experimental import pallas as pl
from jax.experimental.pallas import tpu as pltpu

D = 64
NEG = -1e30


def _mm_body(x_ref, w_ref, b_ref, o_ref):
    o_ref[...] = (
        jnp.dot(x_ref[...], w_ref[...], preferred_element_type=jnp.float32)
        + b_ref[...]
    )


def _tc_matmul(x, w, b, block_rows):
    m, k = x.shape
    n = w.shape[1]
    assert m % block_rows == 0
    grid = (m // block_rows,)
    return pl.pallas_call(
        _mm_body,
        grid=grid,
        in_specs=[
            pl.BlockSpec((block_rows, k), lambda i: (i, 0)),
            pl.BlockSpec((k, n), lambda i: (0, 0)),
            pl.BlockSpec((1, n), lambda i: (0, 0)),
        ],
        out_specs=pl.BlockSpec((block_rows, n), lambda i: (i, 0)),
        out_shape=jax.ShapeDtypeStruct((m, n), jnp.float32),
    )(x, w, b.reshape(1, n))


def _mm2_body(a_ref, c_ref, wa_ref, wc_ref, b_ref, o_ref):
    o_ref[...] = (
        jnp.dot(a_ref[...], wa_ref[...], preferred_element_type=jnp.float32)
        + jnp.dot(c_ref[...], wc_ref[...], preferred_element_type=jnp.float32)
        + b_ref[...]
    )


def _tc_matmul2(a, c, wa, wc, b, block_rows):
    m, k = a.shape
    n = wa.shape[1]
    assert m % block_rows == 0
    grid = (m // block_rows,)
    return pl.pallas_call(
        _mm2_body,
        grid=grid,
        in_specs=[
            pl.BlockSpec((block_rows, k), lambda i: (i, 0)),
            pl.BlockSpec((block_rows, k), lambda i: (i, 0)),
            pl.BlockSpec((k, n), lambda i: (0, 0)),
            pl.BlockSpec((k, n), lambda i: (0, 0)),
            pl.BlockSpec((1, n), lambda i: (0, 0)),
        ],
        out_specs=pl.BlockSpec((block_rows, n), lambda i: (i, 0)),
        out_shape=jax.ShapeDtypeStruct((m, n), jnp.float32),
    )(a, c, wa, wc, b.reshape(1, n))


def _seg_max_excl_self(v, seg, num_seg):
    first = jnp.maximum(jax.ops.segment_max(v, seg, num_segments=num_seg), NEG)
    fmax = jnp.take(first, seg, axis=0)
    is_max = v >= fmax
    masked = jnp.where(is_max, NEG, v)
    second = jnp.maximum(
        jax.ops.segment_max(masked, seg, num_segments=num_seg), NEG
    )
    return jnp.where(is_max, jnp.take(second, seg, axis=0), fmax)


def kernel(ap_feat, ue_feat, edge_feat, edge_index, W1, b1, W2, b2, W3, b3):
    n_ap = ap_feat.shape[0]
    e = edge_feat.shape[0]
    src = edge_index[0]
    dst = edge_index[1]

    w_ap = jnp.concatenate([W1[:D], W2[:D]], axis=1)
    b_ap = jnp.concatenate([jnp.zeros_like(b2), b2])
    w_eb = jnp.concatenate([W1[D:], W2[D:]], axis=1)
    b_eb = jnp.concatenate([b1, jnp.zeros_like(b1)])

    ap_packed = _tc_matmul(ap_feat, w_ap, b_ap, block_rows=2000)
    eb_packed = _tc_matmul(edge_feat, w_eb, b_eb, block_rows=8000)

    ap1 = ap_packed[:, :D]
    ap2b = ap_packed[:, D:]
    eb1 = eb_packed[:, :D]
    zb = eb_packed[:, D:]

    y1 = jnp.take(ap1, src, axis=0) + eb1
    agg_ap = _seg_max_excl_self(y1, src, n_ap)
    excl = _seg_max_excl_self(zb, dst, n_ap)
    agg_ue = jnp.take(ap2b, src, axis=0) + excl
    agg = jnp.maximum(agg_ap, agg_ue)

    return _tc_matmul2(agg, edge_feat, W3[:D], W3[D:], b3, block_rows=8000)


# speedup vs baseline: 1.1551x; 1.1551x over previous
import functools

import jax
import jax.numpy as jnp
from jax import lax
from jax.experimental import pallas as pl
from jax.experimental.pallas import tpu as pltpu
from jax.experimental.pallas import tpu_sc as plsc

D = 64
NEG = -1e30

NW = 32
P = 2
R = 784
NPAD = NW * P * R
CHUNK = 4000
CBUF = 4608
G = 128


def _mm_body(x_ref, w_ref, b_ref, o_ref):
    o_ref[...] = (
        jnp.dot(x_ref[...], w_ref[...], preferred_element_type=jnp.float32)
        + b_ref[...]
    )


def _tc_matmul(x, w, b, block_rows):
    m, k = x.shape
    n = w.shape[1]
    grid = (m // block_rows,)
    return pl.pallas_call(
        _mm_body,
        grid=grid,
        in_specs=[
            pl.BlockSpec((block_rows, k), lambda i: (i, 0)),
            pl.BlockSpec((k, n), lambda i: (0, 0)),
            pl.BlockSpec((1, n), lambda i: (0, 0)),
        ],
        out_specs=pl.BlockSpec((block_rows, n), lambda i: (i, 0)),
        out_shape=jax.ShapeDtypeStruct((m, n), jnp.float32),
    )(x, w, b.reshape(1, n))


def _mm_split_body(x_ref, w_ref, b_ref, o1_ref, o2_ref):
    y = (
        jnp.dot(x_ref[...], w_ref[...], preferred_element_type=jnp.float32)
        + b_ref[...]
    )
    o1_ref[...] = y[:, :D]
    o2_ref[...] = y[:, D:]


def _tc_matmul_split(x, w, b, block_rows):
    m, k = x.shape
    n = w.shape[1]
    grid = (m // block_rows,)
    return pl.pallas_call(
        _mm_split_body,
        grid=grid,
        in_specs=[
            pl.BlockSpec((block_rows, k), lambda i: (i, 0)),
            pl.BlockSpec((k, n), lambda i: (0, 0)),
            pl.BlockSpec((1, n), lambda i: (0, 0)),
        ],
        out_specs=[
            pl.BlockSpec((block_rows, D), lambda i: (i, 0)),
            pl.BlockSpec((block_rows, D), lambda i: (i, 0)),
        ],
        out_shape=[
            jax.ShapeDtypeStruct((m, D), jnp.float32),
            jax.ShapeDtypeStruct((m, D), jnp.float32),
        ],
    )(x, w, b.reshape(1, n))


def _mm2_body(a_ref, c_ref, wa_ref, wc_ref, b_ref, o_ref):
    o_ref[...] = (
        jnp.dot(a_ref[...], wa_ref[...], preferred_element_type=jnp.float32)
        + jnp.dot(c_ref[...], wc_ref[...], preferred_element_type=jnp.float32)
        + b_ref[...]
    )


def _tc_matmul2(a, c, wa, wc, b, block_rows):
    m, k = a.shape
    n = wa.shape[1]
    grid = (m // block_rows,)
    return pl.pallas_call(
        _mm2_body,
        grid=grid,
        in_specs=[
            pl.BlockSpec((block_rows, k), lambda i: (i, 0)),
            pl.BlockSpec((block_rows, k), lambda i: (i, 0)),
            pl.BlockSpec((k, n), lambda i: (0, 0)),
            pl.BlockSpec((k, n), lambda i: (0, 0)),
            pl.BlockSpec((1, n), lambda i: (0, 0)),
        ],
        out_specs=pl.BlockSpec((block_rows, n), lambda i: (i, 0)),
        out_shape=jax.ShapeDtypeStruct((m, n), jnp.float32),
    )(a, c, wa, wc, b.reshape(1, n))


def _seg_kernel_body(ids_hbm, vals_hbm, table_hbm, ids_buf, ebuf, obuf, vbuf,
                     acc, sem):
    c = lax.axis_index("c")
    s = lax.axis_index("s")
    wid = s * 2 + c
    lane = lax.iota(jnp.int32, 16)
    neg16 = jnp.full((16,), NEG, jnp.float32)
    zero16 = jnp.zeros((16,), jnp.int32)
    nids = ids_hbm.shape[0]
    nchunk = nids // CHUNK

    def _init_e(i, carry):
        ebuf[pl.ds(i * 16, 16)] = zero16
        return carry

    lax.fori_loop(0, CBUF // 16, _init_e, 0)

    def _update_group(gbase, nvalid):
        pltpu.async_copy(vals_hbm.at[ebuf.at[pl.ds(gbase, G)]], vbuf,
                         sem).wait()

        def _jj(jj, carry):
            offs = obuf[pl.ds(gbase + jj * 16, 16)]
            offs = jnp.where(lane + jj * 16 < nvalid, offs, R)
            for k in range(16):
                off = offs[k]
                abase = off * 128
                rb = jj * 16 + k
                for fg in range(4):
                    v = vbuf[rb, pl.ds(fg * 16, 16)]
                    m1 = acc[pl.ds(abase + fg * 16, 16)]
                    m2 = acc[pl.ds(abase + 64 + fg * 16, 16)]
                    acc[pl.ds(abase + fg * 16, 16)] = jnp.maximum(m1, v)
                    acc[pl.ds(abase + 64 + fg * 16, 16)] = jnp.maximum(
                        m2, jnp.minimum(m1, v))
            return carry

        lax.fori_loop(0, G // 16, _jj, 0)

    for p in range(P):
        lo = (p * NW + wid) * R

        def _init_acc(i, carry):
            acc[pl.ds(i * 16, 16)] = neg16
            return carry

        lax.fori_loop(0, (R + 1) * 128 // 16, _init_acc, 0)

        def _chunk(ci, cursor):
            pltpu.sync_copy(ids_hbm.at[pl.ds(ci * CHUNK, CHUNK)], ids_buf)

            def _scan(j, cur):
                ids = ids_buf[pl.ds(j * 16, 16)]
                m = (ids >= lo) & (ids < lo + R)
                eidx = ci * CHUNK + j * 16 + lane
                mi = m.astype(jnp.int32)
                cum = plsc.cumsum(mi)
                pos = jnp.where(m, cur + cum - 1, CBUF - 16 + lane)
                plsc.store_scatter(ebuf, [pos], eidx)
                plsc.store_scatter(obuf, [pos], ids - lo)
                return cur + jnp.sum(mi)

            cursor = lax.fori_loop(0, CHUNK // 16, _scan, cursor)
            nfull = cursor // G

            def _g(g, carry):
                _update_group(g * G, G)
                return carry

            lax.fori_loop(0, nfull, _g, 0)
            rem = cursor - nfull * G

            def _mv(i, carry):
                te = ebuf[pl.ds(nfull * G + i * 16, 16)]
                to = obuf[pl.ds(nfull * G + i * 16, 16)]
                ebuf[pl.ds(i * 16, 16)] = te
                obuf[pl.ds(i * 16, 16)] = to
                return carry

            lax.fori_loop(0, (rem + 15) // 16, _mv, 0)
            return rem

        cursor = lax.fori_loop(0, nchunk, _chunk, jnp.int32(0))
        _update_group(0, cursor)
        pltpu.sync_copy(acc.at[pl.ds(0, R * 128)],
                        table_hbm.at[pl.ds(lo * 128, R * 128)])


def _seg_tables(ids, vals):
    mesh = plsc.VectorSubcoreMesh(core_axis_name="c", subcore_axis_name="s")
    out = pl.kernel(
        _seg_kernel_body,
        out_type=jax.ShapeDtypeStruct((NPAD * 128,), jnp.float32),
        mesh=mesh,
        scratch_types=[
            pltpu.VMEM((CHUNK,), jnp.int32),
            pltpu.VMEM((CBUF,), jnp.int32),
            pltpu.VMEM((CBUF,), jnp.int32),
            pltpu.VMEM((G, 64), jnp.float32),
            pltpu.VMEM(((R + 1) * 128,), jnp.float32),
            pltpu.SemaphoreType.DMA,
        ],
        compiler_params=pltpu.CompilerParams(
            needs_layout_passes=False, use_tc_tiling_on_sc=False),
    )(ids, vals)
    return out.reshape(NPAD, 128)


def kernel(ap_feat, ue_feat, edge_feat, edge_index, W1, b1, W2, b2, W3, b3):
    src = edge_index[0]
    dst = edge_index[1]

    w_ap = jnp.concatenate([W1[:D], W2[:D]], axis=1)
    b_ap = jnp.concatenate([jnp.zeros_like(b2), b2])
    w_eb = jnp.concatenate([W1[D:], W2[D:]], axis=1)
    b_eb = jnp.concatenate([b1, jnp.zeros_like(b1)])

    ap_packed = _tc_matmul(ap_feat, w_ap, b_ap, block_rows=2000)
    eb1, zb = _tc_matmul_split(edge_feat, w_eb, b_eb, block_rows=8000)

    ts = _seg_tables(src, eb1)
    td = _seg_tables(dst, zb)

    ap1 = ap_packed[:, :D]
    ap2b = ap_packed[:, D:]

    m1s = jnp.take(ts[:, :D], src, axis=0)
    m2s = jnp.take(ts[:, D:], src, axis=0)
    excl_eb = jnp.where(eb1 >= m1s, m2s, m1s)
    agg_ap = jnp.take(ap1, src, axis=0) + excl_eb

    m1d = jnp.take(td[:, :D], dst, axis=0)
    m2d = jnp.take(td[:, D:], dst, axis=0)
    excl_zb = jnp.where(zb >= m1d, m2d, m1d)
    agg_ue = jnp.take(ap2b, src, axis=0) + excl_zb

    agg = jnp.maximum(agg_ap, agg_ue)
    return _tc_matmul2(agg, edge_feat, W3[:D], W3[D:], b3, block_rows=8000)


# speedup vs baseline: 1.8542x; 1.6052x over previous
import functools

import jax
import jax.numpy as jnp
from jax import lax
from jax.experimental import pallas as pl
from jax.experimental.pallas import tpu as pltpu
from jax.experimental.pallas import tpu_sc as plsc

D = 64
NEG = -1e30

NW = 32
P = 2
R = 784
NPAD = NW * P * R
CHUNK = 4000
CBUF = 4608
G = 128


def _mm_body(x_ref, w_ref, b_ref, o_ref):
    o_ref[...] = (
        jnp.dot(x_ref[...], w_ref[...], preferred_element_type=jnp.float32)
        + b_ref[...]
    )


def _tc_matmul(x, w, b, block_rows):
    m, k = x.shape
    n = w.shape[1]
    grid = (m // block_rows,)
    return pl.pallas_call(
        _mm_body,
        grid=grid,
        in_specs=[
            pl.BlockSpec((block_rows, k), lambda i: (i, 0)),
            pl.BlockSpec((k, n), lambda i: (0, 0)),
            pl.BlockSpec((1, n), lambda i: (0, 0)),
        ],
        out_specs=pl.BlockSpec((block_rows, n), lambda i: (i, 0)),
        out_shape=jax.ShapeDtypeStruct((m, n), jnp.float32),
    )(x, w, b.reshape(1, n))


def _mm_split_body(x_ref, w_ref, b_ref, o1_ref, o2_ref):
    y = (
        jnp.dot(x_ref[...], w_ref[...], preferred_element_type=jnp.float32)
        + b_ref[...]
    )
    o1_ref[...] = y[:, :D]
    o2_ref[...] = y[:, D:]


def _tc_matmul_split(x, w, b, block_rows):
    m, k = x.shape
    n = w.shape[1]
    grid = (m // block_rows,)
    return pl.pallas_call(
        _mm_split_body,
        grid=grid,
        in_specs=[
            pl.BlockSpec((block_rows, k), lambda i: (i, 0)),
            pl.BlockSpec((k, n), lambda i: (0, 0)),
            pl.BlockSpec((1, n), lambda i: (0, 0)),
        ],
        out_specs=[
            pl.BlockSpec((block_rows, D), lambda i: (i, 0)),
            pl.BlockSpec((block_rows, D), lambda i: (i, 0)),
        ],
        out_shape=[
            jax.ShapeDtypeStruct((m, D), jnp.float32),
            jax.ShapeDtypeStruct((m, D), jnp.float32),
        ],
    )(x, w, b.reshape(1, n))


def _mm2_body(a_ref, c_ref, wa_ref, wc_ref, b_ref, o_ref):
    o_ref[...] = (
        jnp.dot(a_ref[...], wa_ref[...], preferred_element_type=jnp.float32)
        + jnp.dot(c_ref[...], wc_ref[...], preferred_element_type=jnp.float32)
        + b_ref[...]
    )


def _tc_matmul2(a, c, wa, wc, b, block_rows):
    m, k = a.shape
    n = wa.shape[1]
    grid = (m // block_rows,)
    return pl.pallas_call(
        _mm2_body,
        grid=grid,
        in_specs=[
            pl.BlockSpec((block_rows, k), lambda i: (i, 0)),
            pl.BlockSpec((block_rows, k), lambda i: (i, 0)),
            pl.BlockSpec((k, n), lambda i: (0, 0)),
            pl.BlockSpec((k, n), lambda i: (0, 0)),
            pl.BlockSpec((1, n), lambda i: (0, 0)),
        ],
        out_specs=pl.BlockSpec((block_rows, n), lambda i: (i, 0)),
        out_shape=jax.ShapeDtypeStruct((m, n), jnp.float32),
    )(a, c, wa, wc, b.reshape(1, n))


def _seg_kernel_body(ids_hbm, vals_hbm, table_hbm, ids_buf, ebuf, obuf, vbuf,
                     acc, sem):
    c = lax.axis_index("c")
    s = lax.axis_index("s")
    wid = s * 2 + c
    lane = lax.iota(jnp.int32, 16)
    neg16 = jnp.full((16,), NEG, jnp.float32)
    zero16 = jnp.zeros((16,), jnp.int32)
    nids = ids_hbm.shape[0]
    nchunk = nids // CHUNK

    def _init_e(i, carry):
        ebuf[pl.ds(i * 16, 16)] = zero16
        return carry

    lax.fori_loop(0, CBUF // 16, _init_e, 0)

    def _update_group(gbase, nvalid):
        pltpu.async_copy(vals_hbm.at[ebuf.at[pl.ds(gbase, G)]], vbuf,
                         sem).wait()

        def _jj(jj, carry):
            offs = obuf[pl.ds(gbase + jj * 16, 16)]
            offs = jnp.where(lane + jj * 16 < nvalid, offs, R)
            for k in range(16):
                off = offs[k]
                abase = off * 128
                rb = jj * 16 + k
                for fg in range(4):
                    v = vbuf[rb, pl.ds(fg * 16, 16)]
                    m1 = acc[pl.ds(abase + fg * 16, 16)]
                    m2 = acc[pl.ds(abase + 64 + fg * 16, 16)]
                    acc[pl.ds(abase + fg * 16, 16)] = jnp.maximum(m1, v)
                    acc[pl.ds(abase + 64 + fg * 16, 16)] = jnp.maximum(
                        m2, jnp.minimum(m1, v))
            return carry

        lax.fori_loop(0, G // 16, _jj, 0)

    for p in range(P):
        lo = (p * NW + wid) * R

        def _init_acc(i, carry):
            acc[pl.ds(i * 16, 16)] = neg16
            return carry

        lax.fori_loop(0, (R + 1) * 128 // 16, _init_acc, 0)

        def _chunk(ci, cursor):
            pltpu.sync_copy(ids_hbm.at[pl.ds(ci * CHUNK, CHUNK)], ids_buf)

            def _scan(j, cur):
                ids = ids_buf[pl.ds(j * 16, 16)]
                m = (ids >= lo) & (ids < lo + R)
                eidx = ci * CHUNK + j * 16 + lane
                mi = m.astype(jnp.int32)
                cum = plsc.cumsum(mi)
                pos = jnp.where(m, cur + cum - 1, CBUF - 16 + lane)
                plsc.store_scatter(ebuf, [pos], eidx)
                plsc.store_scatter(obuf, [pos], ids - lo)
                return cur + jnp.sum(mi)

            cursor = lax.fori_loop(0, CHUNK // 16, _scan, cursor)
            nfull = cursor // G

            def _g(g, carry):
                _update_group(g * G, G)
                return carry

            lax.fori_loop(0, nfull, _g, 0)
            rem = cursor - nfull * G

            def _mv(i, carry):
                te = ebuf[pl.ds(nfull * G + i * 16, 16)]
                to = obuf[pl.ds(nfull * G + i * 16, 16)]
                ebuf[pl.ds(i * 16, 16)] = te
                obuf[pl.ds(i * 16, 16)] = to
                return carry

            lax.fori_loop(0, (rem + 15) // 16, _mv, 0)
            return rem

        cursor = lax.fori_loop(0, nchunk, _chunk, jnp.int32(0))
        _update_group(0, cursor)
        pltpu.sync_copy(acc.at[pl.ds(0, R * 128)],
                        table_hbm.at[pl.ds(lo * 128, R * 128)])


def _seg_tables(ids, vals):
    mesh = plsc.VectorSubcoreMesh(core_axis_name="c", subcore_axis_name="s")
    out = pl.kernel(
        _seg_kernel_body,
        out_type=jax.ShapeDtypeStruct((NPAD * 128,), jnp.float32),
        mesh=mesh,
        scratch_types=[
            pltpu.VMEM((CHUNK,), jnp.int32),
            pltpu.VMEM((CBUF,), jnp.int32),
            pltpu.VMEM((CBUF,), jnp.int32),
            pltpu.VMEM((G, 64), jnp.float32),
            pltpu.VMEM(((R + 1) * 128,), jnp.float32),
            pltpu.SemaphoreType.DMA,
        ],
        compiler_params=pltpu.CompilerParams(
            needs_layout_passes=False, use_tc_tiling_on_sc=False),
    )(ids, vals)
    return out.reshape(NPAD, 128)


def _finish_body(src_hbm, dst_hbm, ap_hbm, eb1_hbm, zb_hbm, ts_hbm, td_hbm,
                 agg_hbm, sbuf, dbuf, apb, tsb, tdb, eb1b, zbb, outb,
                 s0, s1, s2, s3, s4):
    c = lax.axis_index("c")
    s = lax.axis_index("s")
    wid = s * 2 + c
    e = src_hbm.shape[0]
    nchunks = e // G
    extra = nchunks - (nchunks // NW) * NW
    nch = jnp.where(wid < extra, nchunks // NW + 1, nchunks // NW)

    def _chunk(t, carry):
        base = (wid + t * NW) * G
        pltpu.sync_copy(src_hbm.at[pl.ds(base, G)], sbuf)
        pltpu.sync_copy(dst_hbm.at[pl.ds(base, G)], dbuf)
        ca = pltpu.async_copy(ap_hbm.at[sbuf], apb, s0)
        cs = pltpu.async_copy(ts_hbm.at[sbuf], tsb, s1)
        cd = pltpu.async_copy(td_hbm.at[dbuf], tdb, s2)
        ce = pltpu.async_copy(eb1_hbm.at[pl.ds(base, G)], eb1b, s3)
        cz = pltpu.async_copy(zb_hbm.at[pl.ds(base, G)], zbb, s4)
        ca.wait()
        cs.wait()
        cd.wait()
        ce.wait()
        cz.wait()

        def _row(r, carry2):
            for fg in range(4):
                f0 = fg * 16
                eb1v = eb1b[r, pl.ds(f0, 16)]
                m1s = tsb[r, pl.ds(f0, 16)]
                m2s = tsb[r, pl.ds(64 + f0, 16)]
                agg_ap = apb[r, pl.ds(f0, 16)] + jnp.where(
                    eb1v >= m1s, m2s, m1s)
                zbv = zbb[r, pl.ds(f0, 16)]
                m1d = tdb[r, pl.ds(f0, 16)]
                m2d = tdb[r, pl.ds(64 + f0, 16)]
                agg_ue = apb[r, pl.ds(64 + f0, 16)] + jnp.where(
                    zbv >= m1d, m2d, m1d)
                outb[r, pl.ds(f0, 16)] = jnp.maximum(agg_ap, agg_ue)
            return carry2

        lax.fori_loop(0, G, _row, 0)
        pltpu.sync_copy(outb, agg_hbm.at[pl.ds(base, G)])
        return carry

    lax.fori_loop(0, nch, _chunk, 0)


def _finish(src, dst, ap_packed, eb1, zb, ts, td):
    e = src.shape[0]
    mesh = plsc.VectorSubcoreMesh(core_axis_name="c", subcore_axis_name="s")
    return pl.kernel(
        _finish_body,
        out_type=jax.ShapeDtypeStruct((e, D), jnp.float32),
        mesh=mesh,
        scratch_types=[
            pltpu.VMEM((G,), jnp.int32),
            pltpu.VMEM((G,), jnp.int32),
            pltpu.VMEM((G, 128), jnp.float32),
            pltpu.VMEM((G, 128), jnp.float32),
            pltpu.VMEM((G, 128), jnp.float32),
            pltpu.VMEM((G, D), jnp.float32),
            pltpu.VMEM((G, D), jnp.float32),
            pltpu.VMEM((G, D), jnp.float32),
            pltpu.SemaphoreType.DMA,
            pltpu.SemaphoreType.DMA,
            pltpu.SemaphoreType.DMA,
            pltpu.SemaphoreType.DMA,
            pltpu.SemaphoreType.DMA,
        ],
        compiler_params=pltpu.CompilerParams(
            needs_layout_passes=False, use_tc_tiling_on_sc=False),
    )(src, dst, ap_packed, eb1, zb, ts, td)


def kernel(ap_feat, ue_feat, edge_feat, edge_index, W1, b1, W2, b2, W3, b3):
    src = edge_index[0]
    dst = edge_index[1]

    w_ap = jnp.concatenate([W1[:D], W2[:D]], axis=1)
    b_ap = jnp.concatenate([jnp.zeros_like(b2), b2])
    w_eb = jnp.concatenate([W1[D:], W2[D:]], axis=1)
    b_eb = jnp.concatenate([b1, jnp.zeros_like(b1)])

    ap_packed = _tc_matmul(ap_feat, w_ap, b_ap, block_rows=2000)
    eb1, zb = _tc_matmul_split(edge_feat, w_eb, b_eb, block_rows=8000)

    ts = _seg_tables(src, eb1)
    td = _seg_tables(dst, zb)

    agg = _finish(src, dst, ap_packed, eb1, zb, ts, td)
    return _tc_matmul2(agg, edge_feat, W3[:D], W3[D:], b3, block_rows=8000)


# speedup vs baseline: 1.9819x; 1.0688x over previous
import functools

import jax
import jax.numpy as jnp
from jax import lax
from jax.experimental import pallas as pl
from jax.experimental.pallas import tpu as pltpu
from jax.experimental.pallas import tpu_sc as plsc

D = 64
NEG = -1e30

NW = 32
P = 2
R = 784
NPAD = NW * P * R
CHUNK = 4000
CBUF = 4608
G = 128


def _mm_body(x_ref, w_ref, b_ref, o_ref):
    o_ref[...] = (
        jnp.dot(x_ref[...], w_ref[...], preferred_element_type=jnp.float32)
        + b_ref[...]
    )


def _tc_matmul(x, w, b, block_rows):
    m, k = x.shape
    n = w.shape[1]
    grid = (m // block_rows,)
    return pl.pallas_call(
        _mm_body,
        grid=grid,
        in_specs=[
            pl.BlockSpec((block_rows, k), lambda i: (i, 0)),
            pl.BlockSpec((k, n), lambda i: (0, 0)),
            pl.BlockSpec((1, n), lambda i: (0, 0)),
        ],
        out_specs=pl.BlockSpec((block_rows, n), lambda i: (i, 0)),
        out_shape=jax.ShapeDtypeStruct((m, n), jnp.float32),
    )(x, w, b.reshape(1, n))


def _mm_split_body(x_ref, w_ref, b_ref, o1_ref, o2_ref):
    y = (
        jnp.dot(x_ref[...], w_ref[...], preferred_element_type=jnp.float32)
        + b_ref[...]
    )
    o1_ref[...] = y[:, :D]
    o2_ref[...] = y[:, D:]


def _tc_matmul_split(x, w, b, block_rows):
    m, k = x.shape
    n = w.shape[1]
    grid = (m // block_rows,)
    return pl.pallas_call(
        _mm_split_body,
        grid=grid,
        in_specs=[
            pl.BlockSpec((block_rows, k), lambda i: (i, 0)),
            pl.BlockSpec((k, n), lambda i: (0, 0)),
            pl.BlockSpec((1, n), lambda i: (0, 0)),
        ],
        out_specs=[
            pl.BlockSpec((block_rows, D), lambda i: (i, 0)),
            pl.BlockSpec((block_rows, D), lambda i: (i, 0)),
        ],
        out_shape=[
            jax.ShapeDtypeStruct((m, D), jnp.float32),
            jax.ShapeDtypeStruct((m, D), jnp.float32),
        ],
    )(x, w, b.reshape(1, n))


def _mm2_body(a_ref, c_ref, wa_ref, wc_ref, b_ref, o_ref):
    o_ref[...] = (
        jnp.dot(a_ref[...], wa_ref[...], preferred_element_type=jnp.float32)
        + jnp.dot(c_ref[...], wc_ref[...], preferred_element_type=jnp.float32)
        + b_ref[...]
    )


def _tc_matmul2(a, c, wa, wc, b, block_rows):
    m, k = a.shape
    n = wa.shape[1]
    grid = (m // block_rows,)
    return pl.pallas_call(
        _mm2_body,
        grid=grid,
        in_specs=[
            pl.BlockSpec((block_rows, k), lambda i: (i, 0)),
            pl.BlockSpec((block_rows, k), lambda i: (i, 0)),
            pl.BlockSpec((k, n), lambda i: (0, 0)),
            pl.BlockSpec((k, n), lambda i: (0, 0)),
            pl.BlockSpec((1, n), lambda i: (0, 0)),
        ],
        out_specs=pl.BlockSpec((block_rows, n), lambda i: (i, 0)),
        out_shape=jax.ShapeDtypeStruct((m, n), jnp.float32),
    )(a, c, wa, wc, b.reshape(1, n))


def _seg_kernel_body(ids_hbm, vals_hbm, table_hbm, ids_buf, ebuf, obuf, vbuf,
                     acc, sem):
    c = lax.axis_index("c")
    s = lax.axis_index("s")
    wid = s * 2 + c
    lane = lax.iota(jnp.int32, 16)
    neg16 = jnp.full((16,), NEG, jnp.float32)
    zero16 = jnp.zeros((16,), jnp.int32)
    nids = ids_hbm.shape[0]
    nchunk = nids // CHUNK

    def _init_e(i, carry):
        ebuf[pl.ds(i * 16, 16)] = zero16
        return carry

    lax.fori_loop(0, CBUF // 16, _init_e, 0)

    def _update_group(gbase, nvalid, masked):
        pltpu.async_copy(vals_hbm.at[ebuf.at[pl.ds(gbase, G)]], vbuf,
                         sem).wait()

        def _jj(jj, carry):
            offs = obuf[pl.ds(gbase + jj * 16, 16)]
            if masked:
                offs = jnp.where(lane + jj * 16 < nvalid, offs, R)
            for k in range(16):
                off = offs[k]
                abase = off * 128
                rb = jj * 16 + k
                for fg in range(4):
                    v = vbuf[rb, pl.ds(fg * 16, 16)]
                    m1 = acc[pl.ds(abase + fg * 16, 16)]
                    m2 = acc[pl.ds(abase + 64 + fg * 16, 16)]
                    acc[pl.ds(abase + fg * 16, 16)] = jnp.maximum(m1, v)
                    acc[pl.ds(abase + 64 + fg * 16, 16)] = jnp.maximum(
                        m2, jnp.minimum(m1, v))
            return carry

        lax.fori_loop(0, G // 16, _jj, 0)

    for p in range(P):
        lo = (p * NW + wid) * R

        def _init_acc(i, carry):
            acc[pl.ds(i * 16, 16)] = neg16
            return carry

        lax.fori_loop(0, (R + 1) * 128 // 16, _init_acc, 0)

        def _chunk(ci, cursor):
            pltpu.sync_copy(ids_hbm.at[pl.ds(ci * CHUNK, CHUNK)], ids_buf)

            def _scan(j, cur):
                ids_a = ids_buf[pl.ds(j * 32, 16)]
                ids_b = ids_buf[pl.ds(j * 32 + 16, 16)]
                m_a = (ids_a >= lo) & (ids_a < lo + R)
                m_b = (ids_b >= lo) & (ids_b < lo + R)
                cum_a = plsc.cumsum(m_a.astype(jnp.int32))
                cum_b = plsc.cumsum(m_b.astype(jnp.int32))
                tot_a = cum_a[15]
                eidx = ci * CHUNK + j * 32 + lane
                pos_a = jnp.where(m_a, cur + cum_a - 1, CBUF - 16 + lane)
                pos_b = jnp.where(m_b, cur + tot_a + cum_b - 1,
                                  CBUF - 16 + lane)
                plsc.store_scatter(ebuf, [pos_a], eidx)
                plsc.store_scatter(obuf, [pos_a], ids_a - lo)
                plsc.store_scatter(ebuf, [pos_b], eidx + 16)
                plsc.store_scatter(obuf, [pos_b], ids_b - lo)
                return cur + (tot_a + cum_b[15])

            cursor = lax.fori_loop(0, CHUNK // 32, _scan, cursor)
            nfull = cursor // G

            def _g(g, carry):
                _update_group(g * G, G, masked=False)
                return carry

            lax.fori_loop(0, nfull, _g, 0)
            rem = cursor - nfull * G

            def _mv(i, carry):
                te = ebuf[pl.ds(nfull * G + i * 16, 16)]
                to = obuf[pl.ds(nfull * G + i * 16, 16)]
                ebuf[pl.ds(i * 16, 16)] = te
                obuf[pl.ds(i * 16, 16)] = to
                return carry

            lax.fori_loop(0, (rem + 15) // 16, _mv, 0)
            return rem

        cursor = lax.fori_loop(0, nchunk, _chunk, jnp.int32(0))
        _update_group(0, cursor, masked=True)
        pltpu.sync_copy(acc.at[pl.ds(0, R * 128)],
                        table_hbm.at[pl.ds(lo * 128, R * 128)])


def _seg_tables(ids, vals):
    mesh = plsc.VectorSubcoreMesh(core_axis_name="c", subcore_axis_name="s")
    out = pl.kernel(
        _seg_kernel_body,
        out_type=jax.ShapeDtypeStruct((NPAD * 128,), jnp.float32),
        mesh=mesh,
        scratch_types=[
            pltpu.VMEM((CHUNK,), jnp.int32),
            pltpu.VMEM((CBUF,), jnp.int32),
            pltpu.VMEM((CBUF,), jnp.int32),
            pltpu.VMEM((G, 64), jnp.float32),
            pltpu.VMEM(((R + 1) * 128,), jnp.float32),
            pltpu.SemaphoreType.DMA,
        ],
        compiler_params=pltpu.CompilerParams(
            needs_layout_passes=False, use_tc_tiling_on_sc=False),
    )(ids, vals)
    return out.reshape(NPAD, 128)


GF = 64


def _finish_body(src_hbm, dst_hbm, ap_hbm, eb1_hbm, zb_hbm, ts_hbm, td_hbm,
                 agg_hbm, sbuf0, dbuf0, apb0, tsb0, tdb0, eb1b0, zbb0,
                 sbuf1, dbuf1, apb1, tsb1, tdb1, eb1b1, zbb1, outb,
                 s0, s1, s2, s3, s4, s5, s6, s7, s8, s9):
    c = lax.axis_index("c")
    s = lax.axis_index("s")
    wid = s * 2 + c
    e = src_hbm.shape[0]
    nchunks = e // GF
    extra = nchunks - (nchunks // NW) * NW
    nch = jnp.where(wid < extra, nchunks // NW + 1, nchunks // NW)
    npair = nch // 2
    odd = nch - npair * 2

    bufs = ((sbuf0, dbuf0, apb0, tsb0, tdb0, eb1b0, zbb0,
             (s0, s1, s2, s3, s4)),
            (sbuf1, dbuf1, apb1, tsb1, tdb1, eb1b1, zbb1,
             (s5, s6, s7, s8, s9)))

    def _issue(t, b):
        sbuf, dbuf, apb, tsb, tdb, eb1b, zbb, sems = bufs[b]
        base = (wid + t * NW) * GF
        pltpu.sync_copy(src_hbm.at[pl.ds(base, GF)], sbuf)
        pltpu.sync_copy(dst_hbm.at[pl.ds(base, GF)], dbuf)
        pltpu.async_copy(ap_hbm.at[sbuf], apb, sems[0])
        pltpu.async_copy(ts_hbm.at[sbuf], tsb, sems[1])
        pltpu.async_copy(td_hbm.at[dbuf], tdb, sems[2])
        pltpu.async_copy(eb1_hbm.at[pl.ds(base, GF)], eb1b, sems[3])
        pltpu.async_copy(zb_hbm.at[pl.ds(base, GF)], zbb, sems[4])

    def _drain(b):
        sbuf, dbuf, apb, tsb, tdb, eb1b, zbb, sems = bufs[b]
        pltpu.make_async_copy(ap_hbm.at[sbuf], apb, sems[0]).wait()
        pltpu.make_async_copy(ts_hbm.at[sbuf], tsb, sems[1]).wait()
        pltpu.make_async_copy(td_hbm.at[dbuf], tdb, sems[2]).wait()
        pltpu.make_async_copy(eb1_hbm.at[pl.ds(0, GF)], eb1b, sems[3]).wait()
        pltpu.make_async_copy(zb_hbm.at[pl.ds(0, GF)], zbb, sems[4]).wait()

    def _compute(t, b):
        _, _, apb, tsb, tdb, eb1b, zbb, _ = bufs[b]
        base = (wid + t * NW) * GF

        def _row(r2, carry2):
            for rr in range(2):
                r = r2 * 2 + rr
                for fg in range(4):
                    f0 = fg * 16
                    eb1v = eb1b[r, pl.ds(f0, 16)]
                    m1s = tsb[r, pl.ds(f0, 16)]
                    m2s = tsb[r, pl.ds(64 + f0, 16)]
                    agg_ap = apb[r, pl.ds(f0, 16)] + jnp.where(
                        eb1v >= m1s, m2s, m1s)
                    zbv = zbb[r, pl.ds(f0, 16)]
                    m1d = tdb[r, pl.ds(f0, 16)]
                    m2d = tdb[r, pl.ds(64 + f0, 16)]
                    agg_ue = apb[r, pl.ds(64 + f0, 16)] + jnp.where(
                        zbv >= m1d, m2d, m1d)
                    outb[r, pl.ds(f0, 16)] = jnp.maximum(agg_ap, agg_ue)
            return carry2

        lax.fori_loop(0, GF // 2, _row, 0)
        pltpu.sync_copy(outb, agg_hbm.at[pl.ds(base, GF)])

    _issue(0, 0)

    def _pair(tp, carry):
        t0 = tp * 2
        _issue(t0 + 1, 1)
        _drain(0)
        _compute(t0, 0)

        @pl.when(t0 + 2 < nch)
        def _():
            _issue(t0 + 2, 0)

        _drain(1)
        _compute(t0 + 1, 1)
        return carry

    lax.fori_loop(0, npair, _pair, 0)

    @pl.when(odd == 1)
    def _():
        _drain(0)
        _compute(npair * 2, 0)


def _finish(src, dst, ap_packed, eb1, zb, ts, td):
    e = src.shape[0]
    mesh = plsc.VectorSubcoreMesh(core_axis_name="c", subcore_axis_name="s")
    dbl = [
        pltpu.VMEM((GF,), jnp.int32),
        pltpu.VMEM((GF,), jnp.int32),
        pltpu.VMEM((GF, 128), jnp.float32),
        pltpu.VMEM((GF, 128), jnp.float32),
        pltpu.VMEM((GF, 128), jnp.float32),
        pltpu.VMEM((GF, D), jnp.float32),
        pltpu.VMEM((GF, D), jnp.float32),
    ]
    assert GF % 8 == 0 and GF <= 128
    return pl.kernel(
        _finish_body,
        out_type=jax.ShapeDtypeStruct((e, D), jnp.float32),
        mesh=mesh,
        scratch_types=dbl + dbl + [
            pltpu.VMEM((GF, D), jnp.float32),
        ] + [pltpu.SemaphoreType.DMA] * 10,
        compiler_params=pltpu.CompilerParams(
            needs_layout_passes=False, use_tc_tiling_on_sc=False),
    )(src, dst, ap_packed, eb1, zb, ts, td)


def kernel(ap_feat, ue_feat, edge_feat, edge_index, W1, b1, W2, b2, W3, b3):
    src = edge_index[0]
    dst = edge_index[1]

    w_ap = jnp.concatenate([W1[:D], W2[:D]], axis=1)
    b_ap = jnp.concatenate([jnp.zeros_like(b2), b2])
    w_eb = jnp.concatenate([W1[D:], W2[D:]], axis=1)
    b_eb = jnp.concatenate([b1, jnp.zeros_like(b1)])

    ap_packed = _tc_matmul(ap_feat, w_ap, b_ap, block_rows=2000)
    eb1, zb = _tc_matmul_split(edge_feat, w_eb, b_eb, block_rows=8000)

    ts = _seg_tables(src, eb1)
    td = _seg_tables(dst, zb)

    agg = _finish(src, dst, ap_packed, eb1, zb, ts, td)
    return _tc_matmul2(agg, edge_feat, W3[:D], W3[D:], b3, block_rows=8000)


# speedup vs baseline: 2.2244x; 1.1224x over previous
import functools

import jax
import jax.numpy as jnp
from jax import lax
from jax.experimental import pallas as pl
from jax.experimental.pallas import tpu as pltpu
from jax.experimental.pallas import tpu_sc as plsc

D = 64
NEG = -1e30

NW = 32
P = 2
R = 784
NPAD = NW * P * R
CHUNK = 1600
CAP = 4096
G = 128


def _mm_body(x_ref, w_ref, b_ref, o_ref):
    o_ref[...] = (
        jnp.dot(x_ref[...], w_ref[...], preferred_element_type=jnp.float32)
        + b_ref[...]
    )


def _tc_matmul(x, w, b, block_rows):
    m, k = x.shape
    n = w.shape[1]
    grid = (m // block_rows,)
    return pl.pallas_call(
        _mm_body,
        grid=grid,
        in_specs=[
            pl.BlockSpec((block_rows, k), lambda i: (i, 0)),
            pl.BlockSpec((k, n), lambda i: (0, 0)),
            pl.BlockSpec((1, n), lambda i: (0, 0)),
        ],
        out_specs=pl.BlockSpec((block_rows, n), lambda i: (i, 0)),
        out_shape=jax.ShapeDtypeStruct((m, n), jnp.float32),
    )(x, w, b.reshape(1, n))


def _mm_split_body(x_ref, w_ref, b_ref, o1_ref, o2_ref):
    y = (
        jnp.dot(x_ref[...], w_ref[...], preferred_element_type=jnp.float32)
        + b_ref[...]
    )
    o1_ref[...] = y[:, :D]
    o2_ref[...] = y[:, D:]


def _tc_matmul_split(x, w, b, block_rows):
    m, k = x.shape
    n = w.shape[1]
    grid = (m // block_rows,)
    return pl.pallas_call(
        _mm_split_body,
        grid=grid,
        in_specs=[
            pl.BlockSpec((block_rows, k), lambda i: (i, 0)),
            pl.BlockSpec((k, n), lambda i: (0, 0)),
            pl.BlockSpec((1, n), lambda i: (0, 0)),
        ],
        out_specs=[
            pl.BlockSpec((block_rows, D), lambda i: (i, 0)),
            pl.BlockSpec((block_rows, D), lambda i: (i, 0)),
        ],
        out_shape=[
            jax.ShapeDtypeStruct((m, D), jnp.float32),
            jax.ShapeDtypeStruct((m, D), jnp.float32),
        ],
    )(x, w, b.reshape(1, n))


def _mm2_body(a_ref, c_ref, wa_ref, wc_ref, b_ref, o_ref):
    o_ref[...] = (
        jnp.dot(a_ref[...], wa_ref[...], preferred_element_type=jnp.float32)
        + jnp.dot(c_ref[...], wc_ref[...], preferred_element_type=jnp.float32)
        + b_ref[...]
    )


def _tc_matmul2(a, c, wa, wc, b, block_rows):
    m, k = a.shape
    n = wa.shape[1]
    grid = (m // block_rows,)
    return pl.pallas_call(
        _mm2_body,
        grid=grid,
        in_specs=[
            pl.BlockSpec((block_rows, k), lambda i: (i, 0)),
            pl.BlockSpec((block_rows, k), lambda i: (i, 0)),
            pl.BlockSpec((k, n), lambda i: (0, 0)),
            pl.BlockSpec((k, n), lambda i: (0, 0)),
            pl.BlockSpec((1, n), lambda i: (0, 0)),
        ],
        out_specs=pl.BlockSpec((block_rows, n), lambda i: (i, 0)),
        out_shape=jax.ShapeDtypeStruct((m, n), jnp.float32),
    )(a, c, wa, wc, b.reshape(1, n))


def _seg_kernel_body(ids_hbm, vals_hbm, table_hbm, ids_buf, ebuf, obuf, vbuf,
                     acc, sem, isem):
    c = lax.axis_index("c")
    s = lax.axis_index("s")
    wid = s * 2 + c
    lane = lax.iota(jnp.int32, 16)
    neg16 = jnp.full((16,), NEG, jnp.float32)
    zero16 = jnp.zeros((16,), jnp.int32)
    nids = ids_hbm.shape[0]
    nchunk = nids // CHUNK
    MASK = CAP - 1

    def _init_e(i, carry):
        ebuf[pl.ds(i * 16, 16)] = zero16
        return carry

    lax.fori_loop(0, (CAP + 16) // 16, _init_e, 0)

    def _issue(g):
        rbase = pl.multiple_of((g & 1) * G, G)
        ibase = pl.multiple_of((g * G) & MASK, G)
        pltpu.async_copy(
            vals_hbm.at[ebuf.at[pl.ds(ibase, G)]],
            vbuf.at[pl.ds(rbase, G)], sem)

    def _update(g, nvalid, masked):
        rbase = pl.multiple_of((g & 1) * G, G)
        pltpu.make_async_copy(
            vals_hbm.at[ebuf.at[pl.ds(0, G)]],
            vbuf.at[pl.ds(0, G)], sem).wait()
        obase = pl.multiple_of((g * G) & MASK, G)

        def _jj(jj, carry):
            offs = obuf[pl.ds(obase + jj * 16, 16)]
            if masked:
                offs = jnp.where(lane + jj * 16 < nvalid, offs, R)
            for k in range(16):
                off = offs[k]
                abase = off * 128
                rb = rbase + jj * 16 + k
                for fg in range(4):
                    v = vbuf[rb, pl.ds(fg * 16, 16)]
                    m1 = acc[pl.ds(abase + fg * 16, 16)]
                    m2 = acc[pl.ds(abase + 64 + fg * 16, 16)]
                    acc[pl.ds(abase + fg * 16, 16)] = jnp.maximum(m1, v)
                    acc[pl.ds(abase + 64 + fg * 16, 16)] = jnp.maximum(
                        m2, jnp.minimum(m1, v))
            return carry

        lax.fori_loop(0, G // 16, _jj, 0)

    def _ids_issue(ci):
        pltpu.async_copy(
            ids_hbm.at[pl.ds(ci * CHUNK, CHUNK)],
            ids_buf.at[pl.ds((ci & 1) * CHUNK, CHUNK)], isem)

    def _ids_wait():
        pltpu.make_async_copy(
            ids_hbm.at[pl.ds(0, CHUNK)],
            ids_buf.at[pl.ds(0, CHUNK)], isem).wait()

    for p in range(P):
        lo = (p * NW + wid) * R

        def _init_acc(i, carry):
            acc[pl.ds(i * 16, 16)] = neg16
            return carry

        lax.fori_loop(0, (R + 1) * 128 // 16, _init_acc, 0)

        _ids_issue(0)

        def _chunk(ci, cursor):
            ibase = (ci & 1) * CHUNK
            _ids_wait()

            @pl.when(ci + 1 < nchunk)
            def _():
                _ids_issue(ci + 1)

            def _scan(j, cur):
                ids_a = ids_buf[pl.ds(ibase + j * 32, 16)]
                ids_b = ids_buf[pl.ds(ibase + j * 32 + 16, 16)]
                m_a = (ids_a >= lo) & (ids_a < lo + R)
                m_b = (ids_b >= lo) & (ids_b < lo + R)
                cum_a = plsc.cumsum(m_a.astype(jnp.int32))
                cum_b = plsc.cumsum(m_b.astype(jnp.int32))
                tot_a = cum_a[15]
                eidx = ci * CHUNK + j * 32 + lane
                pos_a = jnp.where(m_a, (cur + cum_a - 1) & MASK, CAP + lane)
                pos_b = jnp.where(m_b, (cur + tot_a + cum_b - 1) & MASK,
                                  CAP + lane)
                plsc.store_scatter(ebuf, [pos_a], eidx)
                plsc.store_scatter(obuf, [pos_a], ids_a - lo)
                plsc.store_scatter(ebuf, [pos_b], eidx + 16)
                plsc.store_scatter(obuf, [pos_b], ids_b - lo)
                return cur + (tot_a + cum_b[15])

            prev = cursor
            cursor = lax.fori_loop(0, CHUNK // 32, _scan, cursor)

            def _flush(g, carry):
                @pl.when(g > 0)
                def _():
                    _update(g - 1, G, masked=False)

                _issue(g)
                return carry

            lax.fori_loop(prev // G, cursor // G, _flush, 0)
            return cursor

        cursor = lax.fori_loop(0, nchunk, _chunk, jnp.int32(0))
        glast = cursor // G

        @pl.when(glast > 0)
        def _():
            _update(glast - 1, G, masked=False)

        _issue(glast)
        _update(glast, cursor - glast * G, masked=True)
        pltpu.sync_copy(acc.at[pl.ds(0, R * 128)],
                        table_hbm.at[pl.ds(lo * 128, R * 128)])


def _seg_tables(ids, vals):
    mesh = plsc.VectorSubcoreMesh(core_axis_name="c", subcore_axis_name="s")
    out = pl.kernel(
        _seg_kernel_body,
        out_type=jax.ShapeDtypeStruct((NPAD * 128,), jnp.float32),
        mesh=mesh,
        scratch_types=[
            pltpu.VMEM((2 * CHUNK,), jnp.int32),
            pltpu.VMEM((CAP + 16,), jnp.int32),
            pltpu.VMEM((CAP + 16,), jnp.int32),
            pltpu.VMEM((2 * G, 64), jnp.float32),
            pltpu.VMEM(((R + 1) * 128,), jnp.float32),
            pltpu.SemaphoreType.DMA,
            pltpu.SemaphoreType.DMA,
        ],
        compiler_params=pltpu.CompilerParams(
            needs_layout_passes=False, use_tc_tiling_on_sc=False),
    )(ids, vals)
    return out.reshape(NPAD, 128)


GF = 64


def _finish_body(src_hbm, dst_hbm, ap_hbm, eb1_hbm, zb_hbm, ts_hbm, td_hbm,
                 agg_hbm, sbuf0, dbuf0, apb0, tsb0, tdb0, eb1b0, zbb0,
                 sbuf1, dbuf1, apb1, tsb1, tdb1, eb1b1, zbb1, outb,
                 s0, s1, s2, s3, s4, s5, s6, s7, s8, s9):
    c = lax.axis_index("c")
    s = lax.axis_index("s")
    wid = s * 2 + c
    e = src_hbm.shape[0]
    nchunks = e // GF
    extra = nchunks - (nchunks // NW) * NW
    nch = jnp.where(wid < extra, nchunks // NW + 1, nchunks // NW)
    npair = nch // 2
    odd = nch - npair * 2

    bufs = ((sbuf0, dbuf0, apb0, tsb0, tdb0, eb1b0, zbb0,
             (s0, s1, s2, s3, s4)),
            (sbuf1, dbuf1, apb1, tsb1, tdb1, eb1b1, zbb1,
             (s5, s6, s7, s8, s9)))

    def _issue(t, b):
        sbuf, dbuf, apb, tsb, tdb, eb1b, zbb, sems = bufs[b]
        base = (wid + t * NW) * GF
        pltpu.sync_copy(src_hbm.at[pl.ds(base, GF)], sbuf)
        pltpu.sync_copy(dst_hbm.at[pl.ds(base, GF)], dbuf)
        pltpu.async_copy(ap_hbm.at[sbuf], apb, sems[0])
        pltpu.async_copy(ts_hbm.at[sbuf], tsb, sems[1])
        pltpu.async_copy(td_hbm.at[dbuf], tdb, sems[2])
        pltpu.async_copy(eb1_hbm.at[pl.ds(base, GF)], eb1b, sems[3])
        pltpu.async_copy(zb_hbm.at[pl.ds(base, GF)], zbb, sems[4])

    def _drain(b):
        sbuf, dbuf, apb, tsb, tdb, eb1b, zbb, sems = bufs[b]
        pltpu.make_async_copy(ap_hbm.at[sbuf], apb, sems[0]).wait()
        pltpu.make_async_copy(ts_hbm.at[sbuf], tsb, sems[1]).wait()
        pltpu.make_async_copy(td_hbm.at[dbuf], tdb, sems[2]).wait()
        pltpu.make_async_copy(eb1_hbm.at[pl.ds(0, GF)], eb1b, sems[3]).wait()
        pltpu.make_async_copy(zb_hbm.at[pl.ds(0, GF)], zbb, sems[4]).wait()

    def _compute(t, b):
        _, _, apb, tsb, tdb, eb1b, zbb, _ = bufs[b]
        base = (wid + t * NW) * GF

        def _row(r2, carry2):
            for rr in range(2):
                r = r2 * 2 + rr
                for fg in range(4):
                    f0 = fg * 16
                    eb1v = eb1b[r, pl.ds(f0, 16)]
                    m1s = tsb[r, pl.ds(f0, 16)]
                    m2s = tsb[r, pl.ds(64 + f0, 16)]
                    agg_ap = apb[r, pl.ds(f0, 16)] + jnp.where(
                        eb1v >= m1s, m2s, m1s)
                    zbv = zbb[r, pl.ds(f0, 16)]
                    m1d = tdb[r, pl.ds(f0, 16)]
                    m2d = tdb[r, pl.ds(64 + f0, 16)]
                    agg_ue = apb[r, pl.ds(64 + f0, 16)] + jnp.where(
                        zbv >= m1d, m2d, m1d)
                    outb[r, pl.ds(f0, 16)] = jnp.maximum(agg_ap, agg_ue)
            return carry2

        lax.fori_loop(0, GF // 2, _row, 0)
        pltpu.sync_copy(outb, agg_hbm.at[pl.ds(base, GF)])

    _issue(0, 0)

    def _pair(tp, carry):
        t0 = tp * 2
        _issue(t0 + 1, 1)
        _drain(0)
        _compute(t0, 0)

        @pl.when(t0 + 2 < nch)
        def _():
            _issue(t0 + 2, 0)

        _drain(1)
        _compute(t0 + 1, 1)
        return carry

    lax.fori_loop(0, npair, _pair, 0)

    @pl.when(odd == 1)
    def _():
        _drain(0)
        _compute(npair * 2, 0)


def _finish(src, dst, ap_packed, eb1, zb, ts, td):
    e = src.shape[0]
    mesh = plsc.VectorSubcoreMesh(core_axis_name="c", subcore_axis_name="s")
    dbl = [
        pltpu.VMEM((GF,), jnp.int32),
        pltpu.VMEM((GF,), jnp.int32),
        pltpu.VMEM((GF, 128), jnp.float32),
        pltpu.VMEM((GF, 128), jnp.float32),
        pltpu.VMEM((GF, 128), jnp.float32),
        pltpu.VMEM((GF, D), jnp.float32),
        pltpu.VMEM((GF, D), jnp.float32),
    ]
    assert GF % 8 == 0 and GF <= 128
    return pl.kernel(
        _finish_body,
        out_type=jax.ShapeDtypeStruct((e, D), jnp.float32),
        mesh=mesh,
        scratch_types=dbl + dbl + [
            pltpu.VMEM((GF, D), jnp.float32),
        ] + [pltpu.SemaphoreType.DMA] * 10,
        compiler_params=pltpu.CompilerParams(
            needs_layout_passes=False, use_tc_tiling_on_sc=False),
    )(src, dst, ap_packed, eb1, zb, ts, td)


def kernel(ap_feat, ue_feat, edge_feat, edge_index, W1, b1, W2, b2, W3, b3):
    src = edge_index[0]
    dst = edge_index[1]

    w_ap = jnp.concatenate([W1[:D], W2[:D]], axis=1)
    b_ap = jnp.concatenate([jnp.zeros_like(b2), b2])
    w_eb = jnp.concatenate([W1[D:], W2[D:]], axis=1)
    b_eb = jnp.concatenate([b1, jnp.zeros_like(b1)])

    ap_packed = _tc_matmul(ap_feat, w_ap, b_ap, block_rows=2000)
    eb1, zb = _tc_matmul_split(edge_feat, w_eb, b_eb, block_rows=8000)

    ts = _seg_tables(src, eb1)
    td = _seg_tables(dst, zb)

    agg = _finish(src, dst, ap_packed, eb1, zb, ts, td)
    return _tc_matmul2(agg, edge_feat, W3[:D], W3[D:], b3, block_rows=8000)


# speedup vs baseline: 3.0042x; 1.3506x over previous
import functools

import jax
import jax.numpy as jnp
from jax import lax
from jax.experimental import pallas as pl
from jax.experimental.pallas import tpu as pltpu
from jax.experimental.pallas import tpu_sc as plsc

D = 64
NEG = -1e30

NW = 32
P = 2
R = 784
NPAD = NW * P * R
CHUNK = 1600
CAP = 4096
G = 128


def _mm_t_body(xt_ref, w_ref, b_ref, o1_ref, o2_ref):
    y = lax.dot_general(
        xt_ref[...], w_ref[...], (((0,), (0,)), ((), ())),
        preferred_element_type=jnp.float32) + b_ref[...]
    o1_ref[...] = y[:, :D]
    o2_ref[...] = y[:, D:]


def _tc_matmul_t_split(xt, w, b, block_rows):
    k, m = xt.shape
    n = w.shape[1]
    grid = (m // block_rows,)
    return pl.pallas_call(
        _mm_t_body,
        grid=grid,
        in_specs=[
            pl.BlockSpec((k, block_rows), lambda i: (0, i)),
            pl.BlockSpec((k, n), lambda i: (0, 0)),
            pl.BlockSpec((1, n), lambda i: (0, 0)),
        ],
        out_specs=[
            pl.BlockSpec((block_rows, D), lambda i: (i, 0)),
            pl.BlockSpec((block_rows, D), lambda i: (i, 0)),
        ],
        out_shape=[
            jax.ShapeDtypeStruct((m, D), jnp.float32),
            jax.ShapeDtypeStruct((m, D), jnp.float32),
        ],
    )(xt, w, b.reshape(1, n))


def _mm_t_packed_body(xt_ref, w_ref, b_ref, o_ref):
    o_ref[...] = lax.dot_general(
        xt_ref[...], w_ref[...], (((0,), (0,)), ((), ())),
        preferred_element_type=jnp.float32) + b_ref[...]


def _tc_matmul_t(xt, w, b, block_rows):
    k, m = xt.shape
    n = w.shape[1]
    grid = (m // block_rows,)
    return pl.pallas_call(
        _mm_t_packed_body,
        grid=grid,
        in_specs=[
            pl.BlockSpec((k, block_rows), lambda i: (0, i)),
            pl.BlockSpec((k, n), lambda i: (0, 0)),
            pl.BlockSpec((1, n), lambda i: (0, 0)),
        ],
        out_specs=pl.BlockSpec((block_rows, n), lambda i: (i, 0)),
        out_shape=jax.ShapeDtypeStruct((m, n), jnp.float32),
    )(xt, w, b.reshape(1, n))


def _mm2_t_body(a_ref, ct_ref, wa_ref, wc_ref, b_ref, o_ref):
    ya = lax.dot_general(
        wa_ref[...], a_ref[...], (((0,), (1,)), ((), ())),
        preferred_element_type=jnp.float32)
    yc = lax.dot_general(
        wc_ref[...], ct_ref[...], (((0,), (0,)), ((), ())),
        preferred_element_type=jnp.float32)
    o_ref[...] = ya + yc + b_ref[...]


def _tc_matmul2_t(a, ct, wa, wc, b, block_rows):
    m, k = a.shape
    n = wa.shape[1]
    grid = (m // block_rows,)
    return pl.pallas_call(
        _mm2_t_body,
        grid=grid,
        in_specs=[
            pl.BlockSpec((block_rows, k), lambda i: (i, 0)),
            pl.BlockSpec((k, block_rows), lambda i: (0, i)),
            pl.BlockSpec((k, n), lambda i: (0, 0)),
            pl.BlockSpec((k, n), lambda i: (0, 0)),
            pl.BlockSpec((n, 1), lambda i: (0, 0)),
        ],
        out_specs=pl.BlockSpec((n, block_rows), lambda i: (0, i)),
        out_shape=jax.ShapeDtypeStruct((n, m), jnp.float32),
    )(a, ct, wa, wc, b.reshape(n, 1))


def _seg_kernel_body(ids_hbm, vals_hbm, table_hbm, ids_buf, ebuf, obuf, vbuf,
                     acc, sem, isem):
    c = lax.axis_index("c")
    s = lax.axis_index("s")
    wid = s * 2 + c
    lane = lax.iota(jnp.int32, 16)
    neg16 = jnp.full((16,), NEG, jnp.float32)
    zero16 = jnp.zeros((16,), jnp.int32)
    nids = ids_hbm.shape[0]
    nchunk = nids // CHUNK
    MASK = CAP - 1

    def _init_e(i, carry):
        ebuf[pl.ds(i * 16, 16)] = zero16
        return carry

    lax.fori_loop(0, (CAP + 16) // 16, _init_e, 0)

    def _issue(g):
        rbase = pl.multiple_of((g & 1) * G, G)
        ibase = pl.multiple_of((g * G) & MASK, G)
        pltpu.async_copy(
            vals_hbm.at[ebuf.at[pl.ds(ibase, G)]],
            vbuf.at[pl.ds(rbase, G)], sem)

    def _update(g, nvalid, masked):
        rbase = pl.multiple_of((g & 1) * G, G)
        pltpu.make_async_copy(
            vals_hbm.at[ebuf.at[pl.ds(0, G)]],
            vbuf.at[pl.ds(0, G)], sem).wait()
        obase = pl.multiple_of((g * G) & MASK, G)

        def _jj(jj, carry):
            offs = obuf[pl.ds(obase + jj * 16, 16)]
            if masked:
                offs = jnp.where(lane + jj * 16 < nvalid, offs, R)
            for k in range(16):
                off = offs[k]
                abase = off * 128
                rb = rbase + jj * 16 + k
                for fg in range(4):
                    v = vbuf[rb, pl.ds(fg * 16, 16)]
                    m1 = acc[pl.ds(abase + fg * 16, 16)]
                    m2 = acc[pl.ds(abase + 64 + fg * 16, 16)]
                    acc[pl.ds(abase + fg * 16, 16)] = jnp.maximum(m1, v)
                    acc[pl.ds(abase + 64 + fg * 16, 16)] = jnp.maximum(
                        m2, jnp.minimum(m1, v))
            return carry

        lax.fori_loop(0, G // 16, _jj, 0)

    def _ids_issue(ci):
        pltpu.async_copy(
            ids_hbm.at[pl.ds(ci * CHUNK, CHUNK)],
            ids_buf.at[pl.ds((ci & 1) * CHUNK, CHUNK)], isem)

    def _ids_wait():
        pltpu.make_async_copy(
            ids_hbm.at[pl.ds(0, CHUNK)],
            ids_buf.at[pl.ds(0, CHUNK)], isem).wait()

    for p in range(P):
        lo = (p * NW + wid) * R

        def _init_acc(i, carry):
            acc[pl.ds(i * 16, 16)] = neg16
            return carry

        lax.fori_loop(0, (R + 1) * 128 // 16, _init_acc, 0)

        _ids_issue(0)

        def _chunk(ci, cursor):
            ibase = (ci & 1) * CHUNK
            _ids_wait()

            @pl.when(ci + 1 < nchunk)
            def _():
                _ids_issue(ci + 1)

            def _scan(j, cur):
                idv = [ids_buf[pl.ds(ibase + j * 64 + 16 * q, 16)]
                       for q in range(4)]
                ms = [(v >= lo) & (v < lo + R) for v in idv]
                cums = [plsc.cumsum(m.astype(jnp.int32)) for m in ms]
                base = cur
                eidx = ci * CHUNK + j * 64 + lane
                for q in range(4):
                    pos = jnp.where(ms[q], (base + cums[q] - 1) & MASK,
                                    CAP + lane)
                    plsc.store_scatter(ebuf, [pos], eidx + 16 * q)
                    plsc.store_scatter(obuf, [pos], idv[q] - lo)
                    base = base + cums[q][15]
                return base

            prev = cursor
            cursor = lax.fori_loop(0, CHUNK // 64, _scan, cursor)

            def _flush(g, carry):
                @pl.when(g > 0)
                def _():
                    _update(g - 1, G, masked=False)

                _issue(g)
                return carry

            lax.fori_loop(prev // G, cursor // G, _flush, 0)
            return cursor

        cursor = lax.fori_loop(0, nchunk, _chunk, jnp.int32(0))
        glast = cursor // G

        @pl.when(glast > 0)
        def _():
            _update(glast - 1, G, masked=False)

        _issue(glast)
        _update(glast, cursor - glast * G, masked=True)
        pltpu.sync_copy(acc.at[pl.ds(0, R * 128)],
                        table_hbm.at[pl.ds(lo * 128, R * 128)])


def _seg_tables(ids, vals):
    mesh = plsc.VectorSubcoreMesh(core_axis_name="c", subcore_axis_name="s")
    out = pl.kernel(
        _seg_kernel_body,
        out_type=jax.ShapeDtypeStruct((NPAD * 128,), jnp.float32),
        mesh=mesh,
        scratch_types=[
            pltpu.VMEM((2 * CHUNK,), jnp.int32),
            pltpu.VMEM((CAP + 16,), jnp.int32),
            pltpu.VMEM((CAP + 16,), jnp.int32),
            pltpu.VMEM((2 * G, 64), jnp.float32),
            pltpu.VMEM(((R + 1) * 128,), jnp.float32),
            pltpu.SemaphoreType.DMA,
            pltpu.SemaphoreType.DMA,
        ],
        compiler_params=pltpu.CompilerParams(
            needs_layout_passes=False, use_tc_tiling_on_sc=False),
    )(ids, vals)
    return out.reshape(NPAD, 128)


GF = 64


def _finish_body(src_hbm, dst_hbm, ap_hbm, eb1_hbm, zb_hbm, ts_hbm, td_hbm,
                 agg_hbm, sbuf0, dbuf0, apb0, tsb0, tdb0, eb1b0, zbb0,
                 sbuf1, dbuf1, apb1, tsb1, tdb1, eb1b1, zbb1, outb,
                 s0, s1, s2, s3, s4, s5, s6, s7, s8, s9):
    c = lax.axis_index("c")
    s = lax.axis_index("s")
    wid = s * 2 + c
    e = src_hbm.shape[0]
    nchunks = e // GF
    extra = nchunks - (nchunks // NW) * NW
    nch = jnp.where(wid < extra, nchunks // NW + 1, nchunks // NW)
    npair = nch // 2
    odd = nch - npair * 2

    bufs = ((sbuf0, dbuf0, apb0, tsb0, tdb0, eb1b0, zbb0,
             (s0, s1, s2, s3, s4)),
            (sbuf1, dbuf1, apb1, tsb1, tdb1, eb1b1, zbb1,
             (s5, s6, s7, s8, s9)))

    def _issue(t, b):
        sbuf, dbuf, apb, tsb, tdb, eb1b, zbb, sems = bufs[b]
        base = (wid + t * NW) * GF
        pltpu.sync_copy(src_hbm.at[pl.ds(base, GF)], sbuf)
        pltpu.sync_copy(dst_hbm.at[pl.ds(base, GF)], dbuf)
        pltpu.async_copy(ap_hbm.at[sbuf], apb, sems[0])
        pltpu.async_copy(ts_hbm.at[sbuf], tsb, sems[1])
        pltpu.async_copy(td_hbm.at[dbuf], tdb, sems[2])
        pltpu.async_copy(eb1_hbm.at[pl.ds(base, GF)], eb1b, sems[3])
        pltpu.async_copy(zb_hbm.at[pl.ds(base, GF)], zbb, sems[4])

    def _drain(b):
        sbuf, dbuf, apb, tsb, tdb, eb1b, zbb, sems = bufs[b]
        pltpu.make_async_copy(ap_hbm.at[sbuf], apb, sems[0]).wait()
        pltpu.make_async_copy(ts_hbm.at[sbuf], tsb, sems[1]).wait()
        pltpu.make_async_copy(td_hbm.at[dbuf], tdb, sems[2]).wait()
        pltpu.make_async_copy(eb1_hbm.at[pl.ds(0, GF)], eb1b, sems[3]).wait()
        pltpu.make_async_copy(zb_hbm.at[pl.ds(0, GF)], zbb, sems[4]).wait()

    def _compute(t, b):
        _, _, apb, tsb, tdb, eb1b, zbb, _ = bufs[b]
        base = (wid + t * NW) * GF

        def _row(r2, carry2):
            for rr in range(2):
                r = r2 * 2 + rr
                for fg in range(4):
                    f0 = fg * 16
                    eb1v = eb1b[r, pl.ds(f0, 16)]
                    m1s = tsb[r, pl.ds(f0, 16)]
                    m2s = tsb[r, pl.ds(64 + f0, 16)]
                    agg_ap = apb[r, pl.ds(f0, 16)] + jnp.where(
                        eb1v >= m1s, m2s, m1s)
                    zbv = zbb[r, pl.ds(f0, 16)]
                    m1d = tdb[r, pl.ds(f0, 16)]
                    m2d = tdb[r, pl.ds(64 + f0, 16)]
                    agg_ue = apb[r, pl.ds(64 + f0, 16)] + jnp.where(
                        zbv >= m1d, m2d, m1d)
                    outb[r, pl.ds(f0, 16)] = jnp.maximum(agg_ap, agg_ue)
            return carry2

        lax.fori_loop(0, GF // 2, _row, 0)
        pltpu.sync_copy(outb, agg_hbm.at[pl.ds(base, GF)])

    _issue(0, 0)

    def _pair(tp, carry):
        t0 = tp * 2
        _issue(t0 + 1, 1)
        _drain(0)
        _compute(t0, 0)

        @pl.when(t0 + 2 < nch)
        def _():
            _issue(t0 + 2, 0)

        _drain(1)
        _compute(t0 + 1, 1)
        return carry

    lax.fori_loop(0, npair, _pair, 0)

    @pl.when(odd == 1)
    def _():
        _drain(0)
        _compute(npair * 2, 0)


def _finish(src, dst, ap_packed, eb1, zb, ts, td):
    e = src.shape[0]
    mesh = plsc.VectorSubcoreMesh(core_axis_name="c", subcore_axis_name="s")
    dbl = [
        pltpu.VMEM((GF,), jnp.int32),
        pltpu.VMEM((GF,), jnp.int32),
        pltpu.VMEM((GF, 128), jnp.float32),
        pltpu.VMEM((GF, 128), jnp.float32),
        pltpu.VMEM((GF, 128), jnp.float32),
        pltpu.VMEM((GF, D), jnp.float32),
        pltpu.VMEM((GF, D), jnp.float32),
    ]
    assert GF % 8 == 0 and GF <= 128
    return pl.kernel(
        _finish_body,
        out_type=jax.ShapeDtypeStruct((e, D), jnp.float32),
        mesh=mesh,
        scratch_types=dbl + dbl + [
            pltpu.VMEM((GF, D), jnp.float32),
        ] + [pltpu.SemaphoreType.DMA] * 10,
        compiler_params=pltpu.CompilerParams(
            needs_layout_passes=False, use_tc_tiling_on_sc=False),
    )(src, dst, ap_packed, eb1, zb, ts, td)


def kernel(ap_feat, ue_feat, edge_feat, edge_index, W1, b1, W2, b2, W3, b3):
    src = edge_index[0]
    dst = edge_index[1]

    w_ap = jnp.concatenate([W1[:D], W2[:D]], axis=1)
    b_ap = jnp.concatenate([jnp.zeros_like(b2), b2])
    w_eb = jnp.concatenate([W1[D:], W2[D:]], axis=1)
    b_eb = jnp.concatenate([b1, jnp.zeros_like(b1)])

    ap_t = ap_feat.T
    ef_t = edge_feat.T
    ap_packed = _tc_matmul_t(ap_t, w_ap, b_ap, block_rows=50000)
    eb1, zb = _tc_matmul_t_split(ef_t, w_eb, b_eb, block_rows=6400)

    ts = _seg_tables(src, eb1)
    td = _seg_tables(dst, zb)

    agg = _finish(src, dst, ap_packed, eb1, zb, ts, td)
    out_t = _tc_matmul2_t(agg, ef_t, W3[:D], W3[D:], b3, block_rows=6400)
    return out_t.T
